# Initial kernel scaffold; baseline (speedup 1.0000x reference)
import jax
import jax.numpy as jnp
from jax.experimental import pallas as pl


def kernel(node_states, edge_indices, rel_states, W):
    raise NotImplementedError("write your pallas kernel here")


# speedup vs baseline: 4.5867x; 4.5867x over previous
import functools

import jax
import jax.numpy as jnp
from jax import lax
from jax.experimental import pallas as pl
from jax.experimental.pallas import tpu as pltpu
from jax.experimental.pallas import tpu_sc as plsc

N_NODES_PAD = 10240
CHUNK = 128
SUP = 8


def _sc_scatter(t_hbm, gsrc_hbm, gdst_hbm, cdst_hbm,
                part_a, part_c,
                idx_src_v, idx_dst_v, idx_cnt_v, rows_v, ones_v,
                a_sh, c_sh, sem):
    c = lax.axis_index("c")
    s = lax.axis_index("s")
    wid = c * 16 + s

    n_sup = gsrc_hbm.shape[1] // SUP
    n_csup = cdst_hbm.shape[1] // SUP
    rows_per_sub = N_NODES_PAD // 16

    def _fill_row(i, _):
        for j in range(CHUNK // 16):
            rows_v[i, pl.ds(j * 16, 16)] = jnp.zeros((16,), jnp.float32)
        return 0
    lax.fori_loop(0, CHUNK, _fill_row, 0)
    for j in range(CHUNK // 16):
        ones_v[pl.ds(j * 16, 16)] = jnp.ones((16,), jnp.float32)

    base = s * rows_per_sub
    for k in range(rows_per_sub // CHUNK):
        pltpu.sync_copy(rows_v, a_sh.at[pl.ds(base + k * CHUNK, CHUNK)])
        pltpu.sync_copy(rows_v.at[0], c_sh.at[pl.ds(base + k * CHUNK, CHUNK)])
    plsc.subcore_barrier()

    def _outer(o, _):
        pltpu.sync_copy(gsrc_hbm.at[wid, pl.ds(o * SUP, SUP)], idx_src_v)
        pltpu.sync_copy(gdst_hbm.at[wid, pl.ds(o * SUP, SUP)], idx_dst_v)
        for j in range(SUP):
            pltpu.async_copy(t_hbm.at[idx_src_v.at[j]], rows_v, sem).wait()
            pltpu.sync_copy(rows_v, a_sh.at[idx_dst_v.at[j]], add=True)
        return 0
    lax.fori_loop(0, n_sup, _outer, 0)

    def _couter(o, _):
        pltpu.sync_copy(cdst_hbm.at[wid, pl.ds(o * SUP, SUP)], idx_cnt_v)
        for j in range(SUP):
            pltpu.sync_copy(ones_v, c_sh.at[idx_cnt_v.at[j]], add=True)
        return 0
    lax.fori_loop(0, n_csup, _couter, 0)
    plsc.subcore_barrier()

    pltpu.sync_copy(a_sh.at[pl.ds(base, rows_per_sub)],
                    part_a.at[c, pl.ds(base, rows_per_sub)])
    pltpu.sync_copy(c_sh.at[pl.ds(base, rows_per_sub)],
                    part_c.at[c, pl.ds(base, rows_per_sub)])


def _tc_finish(pa_ref, pc_ref, wt_ref, out_ref):
    x = pa_ref[0] + pa_ref[1]
    y = jnp.dot(x, wt_ref[...], preferred_element_type=jnp.float32)
    cnt = pc_ref[0] + pc_ref[1]
    out_ref[...] = y / cnt[:, None]


def kernel(node_states, edge_indices, rel_states, W):
    batch, n_nodes, comp_dim = node_states.shape
    out_dim = W.shape[0]
    n_edges = edge_indices.shape[1]
    n_rel = rel_states.shape[0]

    t_rows = n_nodes + n_rel + (-(n_nodes + n_rel)) % 8
    table = jnp.zeros((t_rows, comp_dim), jnp.float32)
    table = lax.dynamic_update_slice(table, node_states[0], (0, 0))
    table = lax.dynamic_update_slice(table, -rel_states, (n_nodes, 0))

    dst = edge_indices[1]
    src = edge_indices[2]
    rel = edge_indices[3]

    dummy_dst = n_nodes
    zero_row = n_nodes + n_rel

    n_tasks = 2 * n_edges
    per_w = -(-n_tasks // (32 * CHUNK * SUP)) * CHUNK * SUP
    pad = 32 * per_w - n_tasks
    gsrc = jnp.concatenate([src, n_nodes + rel,
                            jnp.full((pad,), zero_row, jnp.int32)])
    gdst = jnp.concatenate([dst, dst, jnp.full((pad,), dummy_dst, jnp.int32)])
    gsrc = gsrc.reshape(32, per_w // CHUNK, CHUNK)
    gdst = gdst.reshape(32, per_w // CHUNK, CHUNK)

    cper_w = -(-n_edges // (32 * CHUNK * SUP)) * CHUNK * SUP
    cpad = 32 * cper_w - n_edges
    cdst = jnp.concatenate([dst, jnp.full((cpad,), dummy_dst, jnp.int32)])
    cdst = cdst.reshape(32, cper_w // CHUNK, CHUNK)

    mesh = plsc.VectorSubcoreMesh(core_axis_name="c", subcore_axis_name="s")
    sc_call = pl.kernel(
        _sc_scatter,
        out_type=[
            jax.ShapeDtypeStruct((2, N_NODES_PAD, comp_dim), jnp.float32),
            jax.ShapeDtypeStruct((2, N_NODES_PAD), jnp.float32),
        ],
        mesh=mesh,
        scratch_types=[
            pltpu.VMEM((SUP, CHUNK), jnp.int32),
            pltpu.VMEM((SUP, CHUNK), jnp.int32),
            pltpu.VMEM((SUP, CHUNK), jnp.int32),
            pltpu.VMEM((CHUNK, comp_dim), jnp.float32),
            pltpu.VMEM((CHUNK,), jnp.float32),
            pltpu.VMEM_SHARED((N_NODES_PAD, comp_dim), jnp.float32),
            pltpu.VMEM_SHARED((N_NODES_PAD,), jnp.float32),
            pltpu.SemaphoreType.DMA,
        ],
    )
    part_a, part_c = sc_call(table, gsrc, gdst, cdst)

    blk = 1024
    grid = N_NODES_PAD // blk
    out = pl.pallas_call(
        _tc_finish,
        grid=(grid,),
        in_specs=[
            pl.BlockSpec((2, blk, comp_dim), lambda i: (0, i, 0)),
            pl.BlockSpec((2, blk), lambda i: (0, i)),
            pl.BlockSpec((comp_dim, out_dim), lambda i: (0, 0)),
        ],
        out_specs=pl.BlockSpec((blk, out_dim), lambda i: (i, 0)),
        out_shape=jax.ShapeDtypeStruct((N_NODES_PAD, out_dim), jnp.float32),
    )(part_a, part_c, W.T)

    return out[:n_nodes][None]
